# Initial kernel scaffold; baseline (speedup 1.0000x reference)
import jax
import jax.numpy as jnp
from jax.experimental import pallas as pl


def kernel(x, edge_index, Win, bin_, Wg, srelu_bias, Wout, bout):
    raise NotImplementedError("write your pallas kernel here")


# speedup vs baseline: 2.7562x; 2.7562x over previous
import functools

import jax
import jax.numpy as jnp
from jax import lax
from jax.experimental import pallas as pl
from jax.experimental.pallas import tpu as pltpu
from jax.experimental.pallas import tpu_sc as plsc

_N = 10000
_F = 128
_NCLS = 40
_NC = 2
_NS = 16
_E0 = 320000
_EP = 327680
_EPT = _EP // _NS
_CH = 128
_NCHUNK = _EPT // _CH
_HALF = 5000
_HPAD = 5120
_RPT = _HPAD // _NS
_TRASH = _N

_ALPHA = 0.1
_RW = 0.1
_CS = 0.8

_mesh = plsc.VectorSubcoreMesh(core_axis_name="c", subcore_axis_name="s")


def _remap_cols(cidx, c):
    off = c * _HALF
    for j in range(_CH // 16):
        v = cidx[pl.ds(j * 16, 16)]
        local = v - off
        ok = (local >= 0) & (local < _HALF)
        cidx[pl.ds(j * 16, 16)] = jnp.where(ok, local, _HALF)


@functools.partial(
    pl.kernel,
    mesh=_mesh,
    out_type=(
        pltpu.HBM((_HPAD, _F), jnp.float32),
        pltpu.HBM((_HPAD, _F), jnp.float32),
    ),
    scratch_types=[
        pltpu.VMEM_SHARED((_HPAD, _F), jnp.float32),
        pltpu.VMEM((_RPT, _F), jnp.float32),
        pltpu.VMEM((_CH, _F), jnp.float32),
        pltpu.VMEM((_CH,), jnp.int32),
        pltpu.VMEM((_CH,), jnp.int32),
        pltpu.SemaphoreType.DMA,
    ],
)
def _sc_aggregate(row_hbm, col_hbm, g_hbm, zeros_hbm, out0, out1,
                  acc, zbuf, rows_v, ridx, cidx, sem):
    c = lax.axis_index("c")
    s = lax.axis_index("s")
    pltpu.sync_copy(zeros_hbm, zbuf)
    pltpu.sync_copy(zbuf, acc.at[pl.ds(s * _RPT, _RPT)])
    plsc.subcore_barrier()

    def chunk(i, carry):
        base = s * _EPT + i * _CH
        pltpu.sync_copy(row_hbm.at[pl.ds(base, _CH)], ridx)
        pltpu.sync_copy(col_hbm.at[pl.ds(base, _CH)], cidx)
        pltpu.async_copy(g_hbm.at[ridx], rows_v, sem).wait()
        _remap_cols(cidx, c)
        pltpu.sync_copy(rows_v, acc.at[cidx], add=True)
        return carry

    lax.fori_loop(0, _NCHUNK, chunk, 0)
    plsc.subcore_barrier()

    @pl.when(c == 0)
    def _():
        pltpu.sync_copy(acc.at[pl.ds(s * _RPT, _RPT)], out0.at[pl.ds(s * _RPT, _RPT)])

    @pl.when(c == 1)
    def _():
        pltpu.sync_copy(acc.at[pl.ds(s * _RPT, _RPT)], out1.at[pl.ds(s * _RPT, _RPT)])


_BLK = 1000
_HB = _HALF // _BLK


def _half_map(i):
    return (lax.rem(i, _HB), 0)


def _tc_init_body(x_ref, win_ref, bin_ref, d0_ref, d1_ref, h0_ref, g0_ref, dv_ref):
    z = jnp.dot(x_ref[...], win_ref[...], preferred_element_type=jnp.float32)
    h0 = jnp.maximum(z + bin_ref[...], 0.0)
    lower = pl.program_id(0) < _HB
    cnt = jnp.where(lower, d0_ref[...][:, 0:1], d1_ref[...][:, 0:1])
    dv = lax.rsqrt(cnt + 1.0)
    h0_ref[...] = h0
    g0_ref[...] = h0 * dv
    dv_ref[...] = dv


def _tc_init(x, Win, bin_row, d0, d1):
    return pl.pallas_call(
        _tc_init_body,
        grid=(_N // _BLK,),
        in_specs=[
            pl.BlockSpec((_BLK, _F), lambda i: (i, 0)),
            pl.BlockSpec((_F, _F), lambda i: (0, 0)),
            pl.BlockSpec((1, _F), lambda i: (0, 0)),
            pl.BlockSpec((_BLK, _F), _half_map),
            pl.BlockSpec((_BLK, _F), _half_map),
        ],
        out_specs=[
            pl.BlockSpec((_BLK, _F), lambda i: (i, 0)),
            pl.BlockSpec((_BLK, _F), lambda i: (i, 0)),
            pl.BlockSpec((_BLK, 1), lambda i: (i, 0)),
        ],
        out_shape=[
            jax.ShapeDtypeStruct((_N, _F), jnp.float32),
            jax.ShapeDtypeStruct((_N, _F), jnp.float32),
            jax.ShapeDtypeStruct((_N, 1), jnp.float32),
        ],
    )(x, Win, bin_row, d0, d1)


def _tc_layer_body(p0_ref, p1_ref, h_ref, x0_ref, dv_ref, w_ref, b_ref,
                   hn_ref, gn_ref):
    dv = dv_ref[...]
    h = h_ref[...]
    lower = pl.program_id(0) < _HB
    p = jnp.where(lower, p0_ref[...], p1_ref[...])
    agg = p * dv + (dv * dv) * h
    h2 = _CS * agg + _RW * h + _ALPHA * x0_ref[...]
    z = jnp.dot(h2, w_ref[...], preferred_element_type=jnp.float32)
    hn = jnp.maximum(z, b_ref[...])
    hn_ref[...] = hn
    gn_ref[...] = hn * dv


def _tc_layer(p0, p1, h, x0, dv, W, b_row):
    return pl.pallas_call(
        _tc_layer_body,
        grid=(_N // _BLK,),
        in_specs=[
            pl.BlockSpec((_BLK, _F), _half_map),
            pl.BlockSpec((_BLK, _F), _half_map),
            pl.BlockSpec((_BLK, _F), lambda i: (i, 0)),
            pl.BlockSpec((_BLK, _F), lambda i: (i, 0)),
            pl.BlockSpec((_BLK, 1), lambda i: (i, 0)),
            pl.BlockSpec((_F, _F), lambda i: (0, 0)),
            pl.BlockSpec((1, _F), lambda i: (0, 0)),
        ],
        out_specs=[
            pl.BlockSpec((_BLK, _F), lambda i: (i, 0)),
            pl.BlockSpec((_BLK, _F), lambda i: (i, 0)),
        ],
        out_shape=[
            jax.ShapeDtypeStruct((_N, _F), jnp.float32),
            jax.ShapeDtypeStruct((_N, _F), jnp.float32),
        ],
    )(p0, p1, h, x0, dv, W, b_row)


def _tc_out_body(h_ref, w_ref, b_ref, o_ref):
    o_ref[...] = (
        jnp.dot(h_ref[...], w_ref[...], preferred_element_type=jnp.float32)
        + b_ref[...]
    )


def _tc_out(h, Wout, bout_row):
    return pl.pallas_call(
        _tc_out_body,
        grid=(_N // _BLK,),
        in_specs=[
            pl.BlockSpec((_BLK, _F), lambda i: (i, 0)),
            pl.BlockSpec((_F, _NCLS), lambda i: (0, 0)),
            pl.BlockSpec((1, _NCLS), lambda i: (0, 0)),
        ],
        out_specs=pl.BlockSpec((_BLK, _NCLS), lambda i: (i, 0)),
        out_shape=jax.ShapeDtypeStruct((_N, _NCLS), jnp.float32),
    )(h, Wout, bout_row)


def kernel(x, edge_index, Win, bin_, Wg, srelu_bias, Wout, bout):
    npad = _EP - _E0
    row = jnp.concatenate([edge_index[0], jnp.zeros((npad,), jnp.int32)])
    col = jnp.concatenate([edge_index[1], jnp.full((npad,), _TRASH, jnp.int32)])
    zerosF = jnp.zeros((_RPT, _F), jnp.float32)
    onesN = jnp.ones((_N, _F), jnp.float32)

    d0, d1 = _sc_aggregate(row, col, onesN, zerosF)
    h0, g, dv = _tc_init(x, Win, bin_.reshape(1, _F), d0, d1)
    h = h0
    for i in range(Wg.shape[0]):
        p0, p1 = _sc_aggregate(row, col, g, zerosF)
        h, g = _tc_layer(p0, p1, h, h0, dv, Wg[i], srelu_bias[i].reshape(1, _F))
    return _tc_out(h, Wout, bout.reshape(1, _NCLS))


# speedup vs baseline: 3.1924x; 1.1583x over previous
import functools

import jax
import jax.numpy as jnp
from jax import lax
from jax.experimental import pallas as pl
from jax.experimental.pallas import tpu as pltpu
from jax.experimental.pallas import tpu_sc as plsc

_N = 10000
_F = 128
_NCLS = 40
_NC = 2
_NS = 16
_E0 = 320000
_EP = 327680
_EPT = _EP // _NS
_CH = 128
_NCHUNK = _EPT // _CH
_HALF = 5000
_HPAD = 5120
_RPT = _HPAD // _NS
_TRASH = _N

_ALPHA = 0.1
_RW = 0.1
_CS = 0.8

_mesh = plsc.VectorSubcoreMesh(core_axis_name="c", subcore_axis_name="s")


_NBUF = 4
_LOOK = 2


_HCH = _NCHUNK // 2


@functools.partial(
    pl.kernel,
    mesh=_mesh,
    out_type=(
        pltpu.HBM((_HPAD, _F), jnp.float32),
        pltpu.HBM((_HPAD, _F), jnp.float32),
    ),
    scratch_types=[
        pltpu.VMEM_SHARED((_HPAD, _F), jnp.float32),
        pltpu.VMEM((_HCH, _CH), jnp.int32),
        pltpu.VMEM((_HCH, _CH), jnp.int32),
        [pltpu.VMEM((_CH, _F), jnp.float32)] * _NBUF,
        [pltpu.SemaphoreType.DMA] * _NBUF,
        [pltpu.SemaphoreType.DMA] * _NBUF,
    ],
)
def _sc_aggregate(row_hbm, col_hbm, g_hbm, zeros_hbm, out0, out1,
                  acc, ridx, cidx, rows, gsem, ssem):
    c = lax.axis_index("c")
    s = lax.axis_index("s")
    pltpu.sync_copy(zeros_hbm, acc.at[pl.ds(s * _RPT, _RPT)])
    plsc.subcore_barrier()

    off = c * _HALF

    for p in range(2):
        base = s * _NCHUNK + p * _HCH
        pltpu.sync_copy(row_hbm.at[pl.ds(base, _HCH)], ridx)
        pltpu.sync_copy(col_hbm.at[pl.ds(base, _HCH)], cidx)

        def remap(ci, carry):
            for j in range(_CH // 16):
                v = cidx[ci, pl.ds(j * 16, 16)]
                local = v - off
                ok = (local >= 0) & (local < _HALF)
                cidx[ci, pl.ds(j * 16, 16)] = jnp.where(ok, local, _HALF)
            return carry
        lax.fori_loop(0, _HCH, remap, 0)

        for b in range(_LOOK):
            pltpu.async_copy(g_hbm.at[ridx.at[b]], rows[b], gsem[b])

        def group(k, carry):
            for b in range(_NBUF):
                ch = k * _NBUF + b
                b2 = (b + _LOOK) % _NBUF
                ch2 = ch + _LOOK

                @pl.when((ch >= _LOOK) & (ch2 < _HCH))
                def _():
                    pltpu.make_async_copy(rows[b2], acc.at[cidx.at[0]], ssem[b2]).wait()

                @pl.when(ch2 < _HCH)
                def _():
                    pltpu.async_copy(g_hbm.at[ridx.at[ch2]], rows[b2], gsem[b2])

                pltpu.make_async_copy(g_hbm.at[ridx.at[ch]], rows[b], gsem[b]).wait()
                pltpu.async_copy(rows[b], acc.at[cidx.at[ch]], ssem[b], add=True)
            return carry

        lax.fori_loop(0, _HCH // _NBUF, group, 0)
        for b in range(_NBUF):
            pltpu.make_async_copy(rows[b], acc.at[cidx.at[0]], ssem[b]).wait()

    plsc.subcore_barrier()

    @pl.when(c == 0)
    def _():
        pltpu.sync_copy(acc.at[pl.ds(s * _RPT, _RPT)], out0.at[pl.ds(s * _RPT, _RPT)])

    @pl.when(c == 1)
    def _():
        pltpu.sync_copy(acc.at[pl.ds(s * _RPT, _RPT)], out1.at[pl.ds(s * _RPT, _RPT)])


_BLK = 1000
_HB = _HALF // _BLK


def _half_map(i):
    return (lax.rem(i, _HB), 0)


def _tc_init_body(x_ref, win_ref, bin_ref, d0_ref, d1_ref, h0_ref, g0_ref, dv_ref):
    z = jnp.dot(x_ref[...], win_ref[...], preferred_element_type=jnp.float32)
    h0 = jnp.maximum(z + bin_ref[...], 0.0)
    lower = pl.program_id(0) < _HB
    cnt = jnp.where(lower, d0_ref[...][:, 0:1], d1_ref[...][:, 0:1])
    dv = lax.rsqrt(cnt + 1.0)
    h0_ref[...] = h0
    g0_ref[...] = h0 * dv
    dv_ref[...] = dv


def _tc_init(x, Win, bin_row, d0, d1):
    return pl.pallas_call(
        _tc_init_body,
        grid=(_N // _BLK,),
        in_specs=[
            pl.BlockSpec((_BLK, _F), lambda i: (i, 0)),
            pl.BlockSpec((_F, _F), lambda i: (0, 0)),
            pl.BlockSpec((1, _F), lambda i: (0, 0)),
            pl.BlockSpec((_BLK, _F), _half_map),
            pl.BlockSpec((_BLK, _F), _half_map),
        ],
        out_specs=[
            pl.BlockSpec((_BLK, _F), lambda i: (i, 0)),
            pl.BlockSpec((_BLK, _F), lambda i: (i, 0)),
            pl.BlockSpec((_BLK, 1), lambda i: (i, 0)),
        ],
        out_shape=[
            jax.ShapeDtypeStruct((_N, _F), jnp.float32),
            jax.ShapeDtypeStruct((_N, _F), jnp.float32),
            jax.ShapeDtypeStruct((_N, 1), jnp.float32),
        ],
    )(x, Win, bin_row, d0, d1)


def _tc_layer_body(p0_ref, p1_ref, h_ref, x0_ref, dv_ref, w_ref, b_ref,
                   hn_ref, gn_ref):
    dv = dv_ref[...]
    h = h_ref[...]
    lower = pl.program_id(0) < _HB
    p = jnp.where(lower, p0_ref[...], p1_ref[...])
    agg = p * dv + (dv * dv) * h
    h2 = _CS * agg + _RW * h + _ALPHA * x0_ref[...]
    z = jnp.dot(h2, w_ref[...], preferred_element_type=jnp.float32)
    hn = jnp.maximum(z, b_ref[...])
    hn_ref[...] = hn
    gn_ref[...] = hn * dv


def _tc_layer(p0, p1, h, x0, dv, W, b_row):
    return pl.pallas_call(
        _tc_layer_body,
        grid=(_N // _BLK,),
        in_specs=[
            pl.BlockSpec((_BLK, _F), _half_map),
            pl.BlockSpec((_BLK, _F), _half_map),
            pl.BlockSpec((_BLK, _F), lambda i: (i, 0)),
            pl.BlockSpec((_BLK, _F), lambda i: (i, 0)),
            pl.BlockSpec((_BLK, 1), lambda i: (i, 0)),
            pl.BlockSpec((_F, _F), lambda i: (0, 0)),
            pl.BlockSpec((1, _F), lambda i: (0, 0)),
        ],
        out_specs=[
            pl.BlockSpec((_BLK, _F), lambda i: (i, 0)),
            pl.BlockSpec((_BLK, _F), lambda i: (i, 0)),
        ],
        out_shape=[
            jax.ShapeDtypeStruct((_N, _F), jnp.float32),
            jax.ShapeDtypeStruct((_N, _F), jnp.float32),
        ],
    )(p0, p1, h, x0, dv, W, b_row)


def _tc_out_body(h_ref, w_ref, b_ref, o_ref):
    o_ref[...] = (
        jnp.dot(h_ref[...], w_ref[...], preferred_element_type=jnp.float32)
        + b_ref[...]
    )


def _tc_out(h, Wout, bout_row):
    return pl.pallas_call(
        _tc_out_body,
        grid=(_N // _BLK,),
        in_specs=[
            pl.BlockSpec((_BLK, _F), lambda i: (i, 0)),
            pl.BlockSpec((_F, _NCLS), lambda i: (0, 0)),
            pl.BlockSpec((1, _NCLS), lambda i: (0, 0)),
        ],
        out_specs=pl.BlockSpec((_BLK, _NCLS), lambda i: (i, 0)),
        out_shape=jax.ShapeDtypeStruct((_N, _NCLS), jnp.float32),
    )(h, Wout, bout_row)


def kernel(x, edge_index, Win, bin_, Wg, srelu_bias, Wout, bout):
    npad = _EP - _E0
    row = jnp.concatenate([edge_index[0], jnp.zeros((npad,), jnp.int32)])
    col = jnp.concatenate([edge_index[1], jnp.full((npad,), _TRASH, jnp.int32)])
    row = row.reshape(_EP // _CH, _CH)
    col = col.reshape(_EP // _CH, _CH)
    zerosF = jnp.zeros((_RPT, _F), jnp.float32)
    onesN = jnp.ones((_N, _F), jnp.float32)

    d0, d1 = _sc_aggregate(row, col, onesN, zerosF)
    h0, g, dv = _tc_init(x, Win, bin_.reshape(1, _F), d0, d1)
    h = h0
    for i in range(Wg.shape[0]):
        p0, p1 = _sc_aggregate(row, col, g, zerosF)
        h, g = _tc_layer(p0, p1, h, h0, dv, Wg[i], srelu_bias[i].reshape(1, _F))
    return _tc_out(h, Wout, bout.reshape(1, _NCLS))


# speedup vs baseline: 3.4205x; 1.0715x over previous
import functools

import jax
import jax.numpy as jnp
from jax import lax
from jax.experimental import pallas as pl
from jax.experimental.pallas import tpu as pltpu
from jax.experimental.pallas import tpu_sc as plsc

_N = 10000
_F = 128
_NCLS = 40
_NC = 2
_NS = 16
_E0 = 320000
_EP = 327680
_EPT = _EP // _NS
_CH = 128
_NCHUNK = _EPT // _CH
_HALF = 5000
_HPAD = 5120
_RPT = _HPAD // _NS
_TRASH = _N

_ALPHA = 0.1
_RW = 0.1
_CS = 0.8

_mesh = plsc.VectorSubcoreMesh(core_axis_name="c", subcore_axis_name="s")


_NBUF = 4
_LOOK = 2


_HCH = _NCHUNK // 2


@functools.partial(
    pl.kernel,
    mesh=_mesh,
    out_type=(
        pltpu.HBM((_HPAD, _F), jnp.float32),
        pltpu.HBM((_HPAD, _F), jnp.float32),
    ),
    scratch_types=[
        pltpu.VMEM_SHARED((_HPAD, _F), jnp.float32),
        pltpu.VMEM((_HCH, _CH), jnp.int32),
        pltpu.VMEM((_HCH, _CH), jnp.int32),
        [pltpu.VMEM((_CH, _F), jnp.float32)] * _NBUF,
        [pltpu.SemaphoreType.DMA] * _NBUF,
        [pltpu.SemaphoreType.DMA] * _NBUF,
    ],
)
def _sc_aggregate(row_hbm, col_hbm, g_hbm, zeros_hbm, out0, out1,
                  acc, ridx, cidx, rows, gsem, ssem):
    c = lax.axis_index("c")
    s = lax.axis_index("s")
    pltpu.sync_copy(zeros_hbm, acc.at[pl.ds(s * _RPT, _RPT)])
    plsc.subcore_barrier()

    off = c * _HALF

    for p in range(2):
        base = s * _NCHUNK + p * _HCH
        pltpu.sync_copy(row_hbm.at[pl.ds(base, _HCH)], ridx)
        pltpu.sync_copy(col_hbm.at[pl.ds(base, _HCH)], cidx)

        def remap(ci, carry):
            for j in range(_CH // 16):
                v = cidx[ci, pl.ds(j * 16, 16)]
                local = v - off
                ok = (local >= 0) & (local < _HALF)
                trash = _HALF + (v & 63)
                cidx[ci, pl.ds(j * 16, 16)] = jnp.where(ok, local, trash)
            return carry
        lax.fori_loop(0, _HCH, remap, 0)

        for b in range(_LOOK):
            pltpu.async_copy(g_hbm.at[ridx.at[b]], rows[b], gsem[b])

        def group(k, carry):
            for b in range(_NBUF):
                ch = k * _NBUF + b
                b2 = (b + _LOOK) % _NBUF
                ch2 = ch + _LOOK

                @pl.when((ch >= _LOOK) & (ch2 < _HCH))
                def _():
                    pltpu.make_async_copy(rows[b2], acc.at[cidx.at[0]], ssem[b2]).wait()

                @pl.when(ch2 < _HCH)
                def _():
                    pltpu.async_copy(g_hbm.at[ridx.at[ch2]], rows[b2], gsem[b2])

                pltpu.make_async_copy(g_hbm.at[ridx.at[ch]], rows[b], gsem[b]).wait()
                pltpu.async_copy(rows[b], acc.at[cidx.at[ch]], ssem[b], add=True)
            return carry

        lax.fori_loop(0, _HCH // _NBUF, group, 0)
        for b in range(_NBUF):
            pltpu.make_async_copy(rows[b], acc.at[cidx.at[0]], ssem[b]).wait()

    plsc.subcore_barrier()

    @pl.when(c == 0)
    def _():
        pltpu.sync_copy(acc.at[pl.ds(s * _RPT, _RPT)], out0.at[pl.ds(s * _RPT, _RPT)])

    @pl.when(c == 1)
    def _():
        pltpu.sync_copy(acc.at[pl.ds(s * _RPT, _RPT)], out1.at[pl.ds(s * _RPT, _RPT)])


_BLK = 1000
_HB = _HALF // _BLK


def _half_map(i):
    return (lax.rem(i, _HB), 0)


def _tc_init_body(x_ref, win_ref, bin_ref, d0_ref, d1_ref, h0_ref, g0_ref, dv_ref):
    z = jnp.dot(x_ref[...], win_ref[...], preferred_element_type=jnp.float32)
    h0 = jnp.maximum(z + bin_ref[...], 0.0)
    lower = pl.program_id(0) < _HB
    cnt = jnp.where(lower, d0_ref[...][:, 0:1], d1_ref[...][:, 0:1])
    dv = lax.rsqrt(cnt + 1.0)
    h0_ref[...] = h0
    g0_ref[...] = h0 * dv
    dv_ref[...] = dv


def _tc_init(x, Win, bin_row, d0, d1):
    return pl.pallas_call(
        _tc_init_body,
        grid=(_N // _BLK,),
        in_specs=[
            pl.BlockSpec((_BLK, _F), lambda i: (i, 0)),
            pl.BlockSpec((_F, _F), lambda i: (0, 0)),
            pl.BlockSpec((1, _F), lambda i: (0, 0)),
            pl.BlockSpec((_BLK, _F), _half_map),
            pl.BlockSpec((_BLK, _F), _half_map),
        ],
        out_specs=[
            pl.BlockSpec((_BLK, _F), lambda i: (i, 0)),
            pl.BlockSpec((_BLK, _F), lambda i: (i, 0)),
            pl.BlockSpec((_BLK, 1), lambda i: (i, 0)),
        ],
        out_shape=[
            jax.ShapeDtypeStruct((_N, _F), jnp.float32),
            jax.ShapeDtypeStruct((_N, _F), jnp.float32),
            jax.ShapeDtypeStruct((_N, 1), jnp.float32),
        ],
    )(x, Win, bin_row, d0, d1)


def _tc_layer_body(p0_ref, p1_ref, h_ref, x0_ref, dv_ref, w_ref, b_ref,
                   hn_ref, gn_ref):
    dv = dv_ref[...]
    h = h_ref[...]
    lower = pl.program_id(0) < _HB
    p = jnp.where(lower, p0_ref[...], p1_ref[...])
    agg = p * dv + (dv * dv) * h
    h2 = _CS * agg + _RW * h + _ALPHA * x0_ref[...]
    z = jnp.dot(h2, w_ref[...], preferred_element_type=jnp.float32)
    hn = jnp.maximum(z, b_ref[...])
    hn_ref[...] = hn
    gn_ref[...] = hn * dv


def _tc_layer(p0, p1, h, x0, dv, W, b_row):
    return pl.pallas_call(
        _tc_layer_body,
        grid=(_N // _BLK,),
        in_specs=[
            pl.BlockSpec((_BLK, _F), _half_map),
            pl.BlockSpec((_BLK, _F), _half_map),
            pl.BlockSpec((_BLK, _F), lambda i: (i, 0)),
            pl.BlockSpec((_BLK, _F), lambda i: (i, 0)),
            pl.BlockSpec((_BLK, 1), lambda i: (i, 0)),
            pl.BlockSpec((_F, _F), lambda i: (0, 0)),
            pl.BlockSpec((1, _F), lambda i: (0, 0)),
        ],
        out_specs=[
            pl.BlockSpec((_BLK, _F), lambda i: (i, 0)),
            pl.BlockSpec((_BLK, _F), lambda i: (i, 0)),
        ],
        out_shape=[
            jax.ShapeDtypeStruct((_N, _F), jnp.float32),
            jax.ShapeDtypeStruct((_N, _F), jnp.float32),
        ],
    )(p0, p1, h, x0, dv, W, b_row)


def _tc_out_body(h_ref, w_ref, b_ref, o_ref):
    o_ref[...] = (
        jnp.dot(h_ref[...], w_ref[...], preferred_element_type=jnp.float32)
        + b_ref[...]
    )


def _tc_out(h, Wout, bout_row):
    return pl.pallas_call(
        _tc_out_body,
        grid=(_N // _BLK,),
        in_specs=[
            pl.BlockSpec((_BLK, _F), lambda i: (i, 0)),
            pl.BlockSpec((_F, _NCLS), lambda i: (0, 0)),
            pl.BlockSpec((1, _NCLS), lambda i: (0, 0)),
        ],
        out_specs=pl.BlockSpec((_BLK, _NCLS), lambda i: (i, 0)),
        out_shape=jax.ShapeDtypeStruct((_N, _NCLS), jnp.float32),
    )(h, Wout, bout_row)


def kernel(x, edge_index, Win, bin_, Wg, srelu_bias, Wout, bout):
    npad = _EP - _E0
    row = jnp.concatenate([edge_index[0], jnp.zeros((npad,), jnp.int32)])
    col = jnp.concatenate([edge_index[1], jnp.full((npad,), _TRASH, jnp.int32)])
    row = row.reshape(_EP // _CH, _CH)
    col = col.reshape(_EP // _CH, _CH)
    zerosF = jnp.zeros((_RPT, _F), jnp.float32)
    onesN = jnp.ones((_N, _F), jnp.float32)

    d0, d1 = _sc_aggregate(row, col, onesN, zerosF)
    h0, g, dv = _tc_init(x, Win, bin_.reshape(1, _F), d0, d1)
    h = h0
    for i in range(Wg.shape[0]):
        p0, p1 = _sc_aggregate(row, col, g, zerosF)
        h, g = _tc_layer(p0, p1, h, h0, dv, Wg[i], srelu_bias[i].reshape(1, _F))
    return _tc_out(h, Wout, bout.reshape(1, _NCLS))
